# baseline (speedup 1.0000x reference)
import jax
import jax.numpy as jnp
from jax.experimental import pallas as pl
from jax.experimental.pallas import tpu as pltpu

NETS = 100
D = 64
BATCH = 1024
STEP_NETS = 10
GRID = NETS // STEP_NETS
GROUPS = ((0, 4), (4 * D, 4), (8 * D, 2))


def _block_diag(w_stacked, p):
    zeros = jnp.zeros((D, D), dtype=w_stacked.dtype)
    rows = []
    for i in range(p):
        blk = w_stacked[i * D:(i + 1) * D, :]
        row = [blk if j == i else zeros for j in range(p)]
        rows.append(jnp.concatenate(row, axis=1))
    return jnp.concatenate(rows, axis=0)


def _mm(a, b):
    return jax.lax.dot_general(
        a, b, (((1,), (0,)), ((), ())),
        precision=jax.lax.Precision.DEFAULT,
        preferred_element_type=jnp.float32)


def _step(x_ref, w0_ref, w1_ref, w2_ref, out_ref):
    for off, p in GROUPS:
        sl = pl.ds(off, p * D)
        b0 = _block_diag(w0_ref[sl, :], p)
        b1 = _block_diag(w1_ref[sl, :], p)
        b2 = _block_diag(w2_ref[sl, :], p)
        m = _mm(b2, _mm(b1, b0))
        out_ref[sl, :] = _mm(m.astype(jnp.bfloat16),
                             x_ref[sl, :].astype(jnp.bfloat16))


@jax.jit
def kernel(x, w0, w1, w2):
    w0m = w0.reshape(NETS * D, D)
    w1m = w1.reshape(NETS * D, D)
    w2m = w2.reshape(NETS * D, D)
    wspec = pl.BlockSpec((STEP_NETS * D, D), lambda i: (i, 0))
    xspec = pl.BlockSpec((STEP_NETS * D, BATCH), lambda i: (i, 0))
    return pl.pallas_call(
        _step,
        grid=(GRID,),
        in_specs=[xspec, wspec, wspec, wspec],
        out_specs=xspec,
        out_shape=jax.ShapeDtypeStruct((NETS * D, BATCH), jnp.float32),
        compiler_params=pltpu.CompilerParams(
            dimension_semantics=("arbitrary",)),
    )(x, w0m, w1m, w2m)


# speedup vs baseline: 1.0166x; 1.0166x over previous
import jax
import jax.numpy as jnp
from jax.experimental import pallas as pl
from jax.experimental.pallas import tpu as pltpu

NETS = 100
D = 64
BATCH = 1024
P = 4
Q = 5
GRID = NETS // (P * Q)


def _block_diag(w_stacked):
    zeros = jnp.zeros((D, D), dtype=w_stacked.dtype)
    rows = []
    for p in range(P):
        blk = w_stacked[p * D:(p + 1) * D, :]
        row = [blk if q == p else zeros for q in range(P)]
        rows.append(jnp.concatenate(row, axis=1))
    return jnp.concatenate(rows, axis=0)


def _mm(a, b):
    return jax.lax.dot_general(
        a, b, (((1,), (0,)), ((), ())),
        precision=jax.lax.Precision.DEFAULT,
        preferred_element_type=jnp.float32)


def _step(x_ref, w0_ref, w1_ref, w2_ref, out_ref):
    for q in range(Q):
        sl = pl.ds(q * P * D, P * D)
        b0 = _block_diag(w0_ref[sl, :])
        b1 = _block_diag(w1_ref[sl, :])
        b2 = _block_diag(w2_ref[sl, :])
        m = _mm(b2, _mm(b1, b0))
        out_ref[sl, :] = _mm(m.astype(jnp.bfloat16),
                             x_ref[sl, :].astype(jnp.bfloat16))


@jax.jit
def kernel(x, w0, w1, w2):
    w0m = w0.reshape(NETS * D, D)
    w1m = w1.reshape(NETS * D, D)
    w2m = w2.reshape(NETS * D, D)
    wspec = pl.BlockSpec((Q * P * D, D), lambda i: (i, 0))
    xspec = pl.BlockSpec((Q * P * D, BATCH), lambda i: (i, 0))
    return pl.pallas_call(
        _step,
        grid=(GRID,),
        in_specs=[xspec, wspec, wspec, wspec],
        out_specs=xspec,
        out_shape=jax.ShapeDtypeStruct((NETS * D, BATCH), jnp.float32),
        compiler_params=pltpu.CompilerParams(
            dimension_semantics=("arbitrary",)),
    )(x, w0m, w1m, w2m)


# speedup vs baseline: 1.0183x; 1.0017x over previous
import jax
import jax.numpy as jnp
from jax.experimental import pallas as pl
from jax.experimental.pallas import tpu as pltpu

NETS = 100
D = 64
BATCH = 1024
P = 4
Q = 5
GRID = NETS // (P * Q)


def _block_diag(w_stacked):
    zeros = jnp.zeros((D, D), dtype=w_stacked.dtype)
    rows = []
    for p in range(P):
        blk = w_stacked[p * D:(p + 1) * D, :]
        row = [blk if q == p else zeros for q in range(P)]
        rows.append(jnp.concatenate(row, axis=1))
    return jnp.concatenate(rows, axis=0)


def _mm(a, b):
    return jax.lax.dot_general(
        a, b, (((1,), (0,)), ((), ())),
        precision=jax.lax.Precision.DEFAULT,
        preferred_element_type=jnp.float32)


def _step(x_ref, w0_ref, w1_ref, w2_ref, out_ref):
    for q in range(Q):
        sl = pl.ds(q * P * D, P * D)
        b0 = _block_diag(w0_ref[sl, :].astype(jnp.bfloat16))
        b1 = _block_diag(w1_ref[sl, :].astype(jnp.bfloat16))
        b2 = _block_diag(w2_ref[sl, :].astype(jnp.bfloat16))
        m = _mm(b2, _mm(b1, b0).astype(jnp.bfloat16))
        out_ref[sl, :] = _mm(m.astype(jnp.bfloat16),
                             x_ref[sl, :].astype(jnp.bfloat16))


@jax.jit
def kernel(x, w0, w1, w2):
    w0m = w0.reshape(NETS * D, D)
    w1m = w1.reshape(NETS * D, D)
    w2m = w2.reshape(NETS * D, D)
    wspec = pl.BlockSpec((Q * P * D, D), lambda i: (i, 0))
    xspec = pl.BlockSpec((Q * P * D, BATCH), lambda i: (i, 0))
    return pl.pallas_call(
        _step,
        grid=(GRID,),
        in_specs=[xspec, wspec, wspec, wspec],
        out_specs=xspec,
        out_shape=jax.ShapeDtypeStruct((NETS * D, BATCH), jnp.float32),
        compiler_params=pltpu.CompilerParams(
            dimension_semantics=("arbitrary",)),
    )(x, w0m, w1m, w2m)
